# baseline (speedup 1.0000x reference)
import functools

import jax
import jax.numpy as jnp
from jax.experimental import pallas as pl
from jax.experimental.pallas import tpu as pltpu

N = 10000
E = 320000
D = 128
LAYERS = 3

BLK = 1000


def _row_spec():
    return pl.BlockSpec((BLK, D), lambda i: (i, 0))


def _full_spec(shape):
    return pl.BlockSpec(shape, lambda i: tuple(0 for _ in shape))


def _ln(h, g, b):
    mu = jnp.mean(h, axis=-1, keepdims=True)
    var = jnp.mean((h - mu) ** 2, axis=-1, keepdims=True)
    return (h - mu) * jax.lax.rsqrt(var + 1e-5) * g + b


def _pre_body(x_ref, wp_ref, bp_ref, ws_ref, bs_ref, wn_ref, bn_ref,
              h_ref, xself_ref, xn_ref):
    h = jnp.dot(x_ref[...], wp_ref[...], preferred_element_type=jnp.float32) + bp_ref[...]
    h_ref[...] = h
    xself_ref[...] = jnp.dot(h, ws_ref[...], preferred_element_type=jnp.float32) + bs_ref[...]
    xn_ref[...] = jnp.dot(h, wn_ref[...], preferred_element_type=jnp.float32) + bn_ref[...]


def _mid_body(coef_ref, emb_ref, xself_ref, s_ref, mx_ref, deg_ref,
              wc1_ref, wc2_ref, bc_ref, g_ref, bln_ref,
              wl1_ref, wl2_ref, bl_ref,
              ws_ref, bs_ref, wn_ref, bn_ref,
              e_ref, xself2_ref, xn2_ref):
    za0, za1, za2 = coef_ref[0], coef_ref[1], coef_ref[2]
    zc0, zc1 = coef_ref[3], coef_ref[4]
    t_pos, t_neg = coef_ref[5], coef_ref[6]
    zl0, zl1, zl2 = coef_ref[7], coef_ref[8], coef_ref[9]
    s = s_ref[...]
    deg = deg_ref[...]
    mean = s / jnp.maximum(deg, 1.0)
    mx = jnp.where(deg > 0, mx_ref[...], 0.0)
    x_n = za0 * s + za1 * mean + za2 * mx
    xs = xself_ref[...]
    cc = (jnp.dot(xs, wc1_ref[...], preferred_element_type=jnp.float32)
          + jnp.dot(x_n, wc2_ref[...], preferred_element_type=jnp.float32) + bc_ref[...])
    h = zc0 * (xs + x_n) + zc1 * cc
    h = jnp.where(h >= 0, t_pos * h, t_neg * h)
    e = _ln(h, g_ref[...], bln_ref[...])
    e_ref[...] = e
    emb = emb_ref[...]
    lc = (jnp.dot(emb, wl1_ref[...], preferred_element_type=jnp.float32)
          + jnp.dot(e, wl2_ref[...], preferred_element_type=jnp.float32) + bl_ref[...])
    hn = zl0 * e + zl1 * (e + emb) + zl2 * lc
    xself2_ref[...] = jnp.dot(hn, ws_ref[...], preferred_element_type=jnp.float32) + bs_ref[...]
    xn2_ref[...] = jnp.dot(hn, wn_ref[...], preferred_element_type=jnp.float32) + bn_ref[...]


def _fin_body(coef_ref, e0_ref, e1_ref, e2_ref, xself_ref,
              s_ref, mx_ref, deg_ref,
              wc1_ref, wc2_ref, bc_ref, g_ref, bln_ref,
              wa0_ref, wa1_ref, wa2_ref, wa3_ref, ba_ref,
              w1_ref, b1_ref, w2_ref, b2_ref,
              out_ref):
    za0, za1, za2 = coef_ref[0], coef_ref[1], coef_ref[2]
    zc0, zc1 = coef_ref[3], coef_ref[4]
    t_pos, t_neg = coef_ref[5], coef_ref[6]
    zg0, zg1, zg2 = coef_ref[7], coef_ref[8], coef_ref[9]
    s = s_ref[...]
    deg = deg_ref[...]
    mean = s / jnp.maximum(deg, 1.0)
    mx = jnp.where(deg > 0, mx_ref[...], 0.0)
    x_n = za0 * s + za1 * mean + za2 * mx
    xs = xself_ref[...]
    cc = (jnp.dot(xs, wc1_ref[...], preferred_element_type=jnp.float32)
          + jnp.dot(x_n, wc2_ref[...], preferred_element_type=jnp.float32) + bc_ref[...])
    h = zc0 * (xs + x_n) + zc1 * cc
    h = jnp.where(h >= 0, t_pos * h, t_neg * h)
    e3 = _ln(h, g_ref[...], bln_ref[...])
    e0, e1, e2 = e0_ref[...], e1_ref[...], e2_ref[...]
    t = (jnp.dot(e0, wa0_ref[...], preferred_element_type=jnp.float32)
         + jnp.dot(e1, wa1_ref[...], preferred_element_type=jnp.float32)
         + jnp.dot(e2, wa2_ref[...], preferred_element_type=jnp.float32)
         + jnp.dot(e3, wa3_ref[...], preferred_element_type=jnp.float32) + ba_ref[...])
    mmax = jnp.maximum(jnp.maximum(e0, e1), jnp.maximum(e2, e3))
    hagg = zg0 * e3 + zg1 * t + zg2 * mmax
    f1 = jnp.maximum(
        jnp.dot(hagg, w1_ref[...], preferred_element_type=jnp.float32) + b1_ref[...], 0.0)
    out_ref[...] = jnp.dot(f1, w2_ref[...], preferred_element_type=jnp.float32) + b2_ref[...]


def _row_out(n=1):
    sh = jax.ShapeDtypeStruct((N, D), jnp.float32)
    return [sh] * n


_W = lambda: _full_spec((D, D))
_B = lambda: _full_spec((1, D))
_C = lambda: pl.BlockSpec(memory_space=pltpu.SMEM)


def _tc_pre(x, wp, bp, ws, bs, wn, bn):
    return pl.pallas_call(
        _pre_body,
        grid=(N // BLK,),
        in_specs=[_row_spec(), _W(), _B(), _W(), _B(), _W(), _B()],
        out_specs=[_row_spec()] * 3,
        out_shape=_row_out(3),
    )(x, wp, bp, ws, bs, wn, bn)


def _tc_mid(coef, emb, xself, s, mx, deg, wc1, wc2, bc, g, bln,
            wl1, wl2, bl, ws, bs, wn, bn):
    return pl.pallas_call(
        _mid_body,
        grid=(N // BLK,),
        in_specs=[_C(), _row_spec(), _row_spec(), _row_spec(), _row_spec(),
                  pl.BlockSpec((BLK, 1), lambda i: (i, 0)),
                  _W(), _W(), _B(), _B(), _B(),
                  _W(), _W(), _B(),
                  _W(), _B(), _W(), _B()],
        out_specs=[_row_spec()] * 3,
        out_shape=_row_out(3),
    )(coef, emb, xself, s, mx, deg, wc1, wc2, bc, g, bln,
      wl1, wl2, bl, ws, bs, wn, bn)


def _tc_fin(coef, e0, e1, e2, xself, s, mx, deg, wc1, wc2, bc, g, bln,
            wa, ba, w1, b1, w2, b2):
    return pl.pallas_call(
        _fin_body,
        grid=(N // BLK,),
        in_specs=[_C(), _row_spec(), _row_spec(), _row_spec(), _row_spec(),
                  _row_spec(), _row_spec(),
                  pl.BlockSpec((BLK, 1), lambda i: (i, 0)),
                  _W(), _W(), _B(), _B(), _B(),
                  _W(), _W(), _W(), _W(), _B(),
                  _W(), _B(), _W(), _B()],
        out_specs=[_row_spec()],
        out_shape=_row_out(1),
    )(coef, e0, e1, e2, xself, s, mx, deg, wc1, wc2, bc, g, bln,
      wa[0], wa[1], wa[2], wa[3], ba, w1, b1, w2, b2)


from jax import lax
from jax.experimental.pallas import tpu_sc as plsc

NW = 32
NPT = 320
NROWS = 328
NPAD = NW * NPT
SCH = 2048
NFULL = E // SCH
TAIL = E - NFULL * SCH
FCH = 2048
SELCAP = FCH + SCH + 16
CAPW = E + FCH
CH = 128
TRASHW = NPT << 14

_SC_PARAMS = pltpu.CompilerParams(needs_layout_passes=False)


@functools.lru_cache(maxsize=None)
def _index_kernel():
    mesh = plsc.VectorSubcoreMesh(core_axis_name="c", subcore_axis_name="s")

    @functools.partial(
        pl.kernel, mesh=mesh,
        out_type=[jax.ShapeDtypeStruct((NW, CAPW), jnp.int32),
                  jax.ShapeDtypeStruct((NW, 16), jnp.int32)],
        compiler_params=_SC_PARAMS,
        scratch_types=[pltpu.VMEM((SELCAP,), jnp.int32),
                       pltpu.VMEM((SCH,), jnp.int32),
                       pltpu.VMEM((SCH,), jnp.int32),
                       pltpu.VMEM((16,), jnp.int32)])
    def idx_kernel(ei, lists, counts, sel, dbuf, sbuf, obuf):
        wid = lax.axis_index("s") * 2 + lax.axis_index("c")
        lo = wid * NPT
        trash = jnp.full((16,), TRASHW, jnp.int32)

        def fill_trash(g, _):
            sel[pl.ds(g * 16, 16)] = trash
            return 0

        lax.fori_loop(0, SELCAP // 16, fill_trash, 0)

        def scan_chunk(carry, off, ngroups):
            cnt, flushed = carry
            ne = ngroups * 16
            pltpu.sync_copy(ei.at[1, pl.ds(off, ne)], dbuf.at[pl.ds(0, ne)])
            pltpu.sync_copy(ei.at[0, pl.ds(off, ne)], sbuf.at[pl.ds(0, ne)])

            def group(g, cnt):
                d16 = dbuf[pl.ds(g * 16, 16)]
                s16 = sbuf[pl.ds(g * 16, 16)]
                m = (d16 >= lo) & (d16 < lo + NPT)
                word = ((d16 - lo) << 14) | s16
                plsc.store_compressed(sel.at[pl.ds(cnt, 16)], word, mask=m)
                pc = plsc.all_reduce_population_count(m)
                return cnt + pc[0]

            cnt = lax.fori_loop(0, ngroups, group, cnt, unroll=4)

            def do_flush(args):
                cnt, flushed = args
                fo = pl.multiple_of(flushed, FCH)
                pltpu.sync_copy(sel.at[pl.ds(0, FCH)],
                                lists.at[wid, pl.ds(fo, FCH)])

                def shift(g, _):
                    sel[pl.ds(g * 16, 16)] = sel[pl.ds(FCH + g * 16, 16)]
                    return 0

                lax.fori_loop(0, (SELCAP - FCH) // 16, shift, 0)

                def refill(g, _):
                    sel[pl.ds(g * 16, 16)] = trash
                    return 0

                lax.fori_loop((SELCAP - FCH) // 16, SELCAP // 16, refill, 0)
                return (cnt - FCH, flushed + FCH)

            return lax.cond(cnt >= FCH, do_flush, lambda a: a, (cnt, flushed))

        carry = (jnp.int32(0), jnp.int32(0))

        def chunk_body(c, carry):
            return scan_chunk(carry, pl.multiple_of(c * SCH, SCH), SCH // 16)

        carry = lax.fori_loop(0, NFULL, chunk_body, carry)
        carry = scan_chunk(carry, NFULL * SCH, TAIL // 16)
        cnt, flushed = carry
        fo = pl.multiple_of(flushed, FCH)
        pltpu.sync_copy(sel.at[pl.ds(0, FCH)], lists.at[wid, pl.ds(fo, FCH)])
        obuf[...] = jnp.full((16,), 1, jnp.int32) * (flushed + cnt)
        pltpu.sync_copy(obuf, counts.at[wid])

    return idx_kernel


NFC = D // 16


@functools.lru_cache(maxsize=None)
def _agg_kernel(with_deg):
    mesh = plsc.VectorSubcoreMesh(core_axis_name="c", subcore_axis_name="s")
    outs = [jax.ShapeDtypeStruct((NPAD * D,), jnp.float32),
            jax.ShapeDtypeStruct((NPAD * D,), jnp.float32)]
    if with_deg:
        outs.append(jax.ShapeDtypeStruct((NPAD,), jnp.float32))
    scr = [pltpu.VMEM((NROWS * D,), jnp.float32),
           pltpu.VMEM((NROWS * D,), jnp.float32),
           pltpu.VMEM((336,), jnp.float32),
           pltpu.VMEM((336,), jnp.int32),
           pltpu.VMEM((CH,), jnp.int32), pltpu.VMEM((CH,), jnp.int32),
           pltpu.VMEM((CH,), jnp.int32), pltpu.VMEM((CH,), jnp.int32),
           pltpu.VMEM((CH,), jnp.int32), pltpu.VMEM((CH,), jnp.int32),
           pltpu.VMEM((CH, D), jnp.float32), pltpu.VMEM((CH, D), jnp.float32),
           pltpu.VMEM((16,), jnp.int32),
           pltpu.SemaphoreType.DMA, pltpu.SemaphoreType.DMA]

    @functools.partial(pl.kernel, mesh=mesh, out_type=outs,
                       compiler_params=_SC_PARAMS, scratch_types=scr)
    def agg(xn, lists, counts, *refs):
        if with_deg:
            s_hbm, m_hbm, deg_hbm = refs[:3]
            rest = refs[3:]
        else:
            s_hbm, m_hbm = refs[:2]
            deg_hbm = None
            rest = refs[2:]
        (acc_s, acc_m, degacc, tmpi, wbuf0, wbuf1, sbuf0, sbuf1, kbuf0,
         kbuf1, rows0, rows1, cbuf, sem0, sem1) = rest
        bufs = ((wbuf0, sbuf0, kbuf0, rows0, sem0),
                (wbuf1, sbuf1, kbuf1, rows1, sem1))
        wid = lax.axis_index("s") * 2 + lax.axis_index("c")
        zeros = jnp.zeros((16,), jnp.float32)
        neg = jnp.full((16,), -3.0e38, jnp.float32)
        pltpu.sync_copy(counts.at[wid], cbuf)
        cnt = cbuf[...][0]
        nch = (cnt + CH - 1) // CH

        def init(i, _):
            b = i * D
            for f in range(NFC):
                acc_s[pl.ds(b + f * 16, 16)] = zeros
                acc_m[pl.ds(b + f * 16, 16)] = neg
            return 0

        lax.fori_loop(0, NROWS, init, 0)
        if with_deg:
            def initd(i, _):
                degacc[pl.ds(i * 16, 16)] = zeros
                return 0
            lax.fori_loop(0, 336 // 16, initd, 0)

        def load_chunk(c, b):
            wbuf, sbuf, kbuf, rows, sem = bufs[b]
            off = pl.multiple_of(c * CH, CH)
            pltpu.sync_copy(lists.at[wid, pl.ds(off, CH)], wbuf)

            def unpack(g, _):
                w16 = wbuf[pl.ds(g * 16, 16)]
                sbuf[pl.ds(g * 16, 16)] = w16 & 16383
                kbuf[pl.ds(g * 16, 16)] = lax.shift_right_logical(w16, 14)
                return 0

            lax.fori_loop(0, CH // 16, unpack, 0, unroll=2)
            pltpu.async_copy(xn.at[sbuf], rows, sem)

        def accum(b):
            _, sbuf, kbuf, rows, sem = bufs[b]
            pltpu.make_async_copy(xn.at[sbuf], rows, sem).wait()

            iota = lax.iota(jnp.int32, 16)
            ones = jnp.full((16,), 1.0, jnp.float32)

            def group(g, _):
                dvec = kbuf[pl.ds(g * 16, 16)]
                base = g * 16
                plsc.store_scatter(tmpi, [dvec], iota)
                win = plsc.load_gather(tmpi, [dvec]) == iota
                ndup = plsc.all_reduce_population_count(win)
                if with_deg:
                    plsc.addupdate_scatter(degacc, [dvec], ones)
                evec = base + iota
                abase = dvec * D

                def fast():
                    def fblk(fb, _):
                        cb = fb * 16
                        for k in range(16):
                            col = lax.broadcast(cb + k, (16,))
                            vals = plsc.load_gather(rows, [evec, col])
                            plsc.addupdate_scatter(acc_s, [abase + (cb + k)],
                                                   vals)
                            cur = plsc.load_gather(acc_m, [abase + (cb + k)])
                            plsc.store_scatter(acc_m, [abase + (cb + k)],
                                               jnp.maximum(cur, vals))
                        return 0

                    lax.fori_loop(0, NFC, fblk, 0)
                    return None

                def slow():
                    for lane in range(16):
                        dl = dvec[lane]
                        db = dl * D
                        for f in range(NFC):
                            fs = pl.ds(db + f * 16, 16)
                            r = rows[base + lane, pl.ds(f * 16, 16)]
                            acc_s[fs] = acc_s[fs] + r
                            acc_m[fs] = jnp.maximum(acc_m[fs], r)
                    return None

                lax.cond(ndup[0] == 16, fast, slow)
                return 0

            lax.fori_loop(0, CH // 16, group, 0)

        pl.when(nch > 0)(lambda: load_chunk(0, 0))

        def pair(k2, _):
            c0 = k2 * 2
            pl.when(c0 + 1 < nch)(lambda: load_chunk(c0 + 1, 1))
            accum(0)
            pl.when(c0 + 2 < nch)(lambda: load_chunk(c0 + 2, 0))
            pl.when(c0 + 1 < nch)(lambda: accum(1))
            return 0

        lax.fori_loop(0, (nch + 1) // 2, pair, 0)
        ro = pl.multiple_of(wid * (NPT * D), NPT * D)
        pltpu.sync_copy(acc_s.at[pl.ds(0, NPT * D)], s_hbm.at[pl.ds(ro, NPT * D)])
        pltpu.sync_copy(acc_m.at[pl.ds(0, NPT * D)], m_hbm.at[pl.ds(ro, NPT * D)])
        if with_deg:
            rd = pl.multiple_of(wid * NPT, NPT)
            pltpu.sync_copy(degacc.at[pl.ds(0, NPT)],
                            deg_hbm.at[pl.ds(rd, NPT)])

    return agg


def kernel(x, edge_index, batch, params):
    src, dst = edge_index[0], edge_index[1]
    Z = params["Z"]
    P = params

    def b2(p):
        return p["b"].reshape(1, D)

    h0, xself, xn = _tc_pre(
        x, P["preprocess"]["W"], b2(P["preprocess"]),
        P["linears_self"][0]["W"], b2(P["linears_self"][0]),
        P["linears"][0]["W"], b2(P["linears"][0]))

    lists, counts = _index_kernel()(edge_index)
    deg = None
    emb = [h0]
    for i in range(LAYERS):
        if i == 0:
            s, mx, degp = _agg_kernel(True)(xn, lists, counts)
            deg = degp[:N].reshape(N, 1)
        else:
            s, mx = _agg_kernel(False)(xn, lists, counts)
        s = s.reshape(NPAD, D)[:N]
        mx = mx.reshape(NPAD, D)[:N]
        za = Z["agg"][i]
        zc = Z["combine"][i]
        zt = Z["act"][i]
        a = P["prelu"][i]
        wc = P["combine_merger"][i]["W"]
        wc1, wc2 = wc[:D], wc[D:]
        ln = P["layer_norms"][i]
        if i < LAYERS - 1:
            zl = Z["layer_connect"][i]
            coef = jnp.stack([za[0], za[1], za[2], zc[0], zc[1],
                              zt[0] + zt[1], zt[1] * a, zl[0], zl[1], zl[2]])
            wl = P["layer_connect_merger"][i]["W"]
            e, xself, xn = _tc_mid(
                coef, emb[i], xself, s, mx, deg,
                wc1, wc2, b2(P["combine_merger"][i]),
                ln["g"].reshape(1, D), ln["b"].reshape(1, D),
                wl[:D], wl[D:], b2(P["layer_connect_merger"][i]),
                P["linears_self"][i + 1]["W"], b2(P["linears_self"][i + 1]),
                P["linears"][i + 1]["W"], b2(P["linears"][i + 1]))
            emb.append(e)
        else:
            zg = Z["layer_agg"][0]
            coef = jnp.stack([za[0], za[1], za[2], zc[0], zc[1],
                              zt[0] + zt[1], zt[1] * a, zg[0], zg[1], zg[2]])
            wa = P["layer_agg_merger"]["W"]
            was = [wa[k * D:(k + 1) * D] for k in range(LAYERS + 1)]
            (out,) = _tc_fin(
                coef, emb[0], emb[1], emb[2], xself, s, mx, deg,
                wc1, wc2, b2(P["combine_merger"][i]),
                ln["g"].reshape(1, D), ln["b"].reshape(1, D),
                was, b2(P["layer_agg_merger"]),
                P["ffn1"]["W"], b2(P["ffn1"]),
                P["ffn2"]["W"], b2(P["ffn2"]))
    return out


# speedup vs baseline: 3.8272x; 3.8272x over previous
import functools

import jax
import jax.numpy as jnp
from jax.experimental import pallas as pl
from jax.experimental.pallas import tpu as pltpu

N = 10000
E = 320000
D = 128
LAYERS = 3

BLK = 1000


def _row_spec():
    return pl.BlockSpec((BLK, D), lambda i: (i, 0))


def _full_spec(shape):
    return pl.BlockSpec(shape, lambda i: tuple(0 for _ in shape))


def _ln(h, g, b):
    mu = jnp.mean(h, axis=-1, keepdims=True)
    var = jnp.mean((h - mu) ** 2, axis=-1, keepdims=True)
    return (h - mu) * jax.lax.rsqrt(var + 1e-5) * g + b


def _pre_body(x_ref, wp_ref, bp_ref, ws_ref, bs_ref, wn_ref, bn_ref,
              h_ref, xself_ref, xn_ref):
    h = jnp.dot(x_ref[...], wp_ref[...], preferred_element_type=jnp.float32) + bp_ref[...]
    h_ref[...] = h
    xself_ref[...] = jnp.dot(h, ws_ref[...], preferred_element_type=jnp.float32) + bs_ref[...]
    xn_ref[...] = jnp.dot(h, wn_ref[...], preferred_element_type=jnp.float32) + bn_ref[...]


def _mid_body(coef_ref, emb_ref, xself_ref, s_ref, mx_ref, deg_ref,
              wc1_ref, wc2_ref, bc_ref, g_ref, bln_ref,
              wl1_ref, wl2_ref, bl_ref,
              ws_ref, bs_ref, wn_ref, bn_ref,
              e_ref, xself2_ref, xn2_ref):
    za0, za1, za2 = coef_ref[0], coef_ref[1], coef_ref[2]
    zc0, zc1 = coef_ref[3], coef_ref[4]
    t_pos, t_neg = coef_ref[5], coef_ref[6]
    zl0, zl1, zl2 = coef_ref[7], coef_ref[8], coef_ref[9]
    s = s_ref[...]
    deg = deg_ref[...]
    mean = s / jnp.maximum(deg, 1.0)
    mx = jnp.where(deg > 0, mx_ref[...], 0.0)
    x_n = za0 * s + za1 * mean + za2 * mx
    xs = xself_ref[...]
    cc = (jnp.dot(xs, wc1_ref[...], preferred_element_type=jnp.float32)
          + jnp.dot(x_n, wc2_ref[...], preferred_element_type=jnp.float32) + bc_ref[...])
    h = zc0 * (xs + x_n) + zc1 * cc
    h = jnp.where(h >= 0, t_pos * h, t_neg * h)
    e = _ln(h, g_ref[...], bln_ref[...])
    e_ref[...] = e
    emb = emb_ref[...]
    lc = (jnp.dot(emb, wl1_ref[...], preferred_element_type=jnp.float32)
          + jnp.dot(e, wl2_ref[...], preferred_element_type=jnp.float32) + bl_ref[...])
    hn = zl0 * e + zl1 * (e + emb) + zl2 * lc
    xself2_ref[...] = jnp.dot(hn, ws_ref[...], preferred_element_type=jnp.float32) + bs_ref[...]
    xn2_ref[...] = jnp.dot(hn, wn_ref[...], preferred_element_type=jnp.float32) + bn_ref[...]


def _fin_body(coef_ref, e0_ref, e1_ref, e2_ref, xself_ref,
              s_ref, mx_ref, deg_ref,
              wc1_ref, wc2_ref, bc_ref, g_ref, bln_ref,
              wa0_ref, wa1_ref, wa2_ref, wa3_ref, ba_ref,
              w1_ref, b1_ref, w2_ref, b2_ref,
              out_ref):
    za0, za1, za2 = coef_ref[0], coef_ref[1], coef_ref[2]
    zc0, zc1 = coef_ref[3], coef_ref[4]
    t_pos, t_neg = coef_ref[5], coef_ref[6]
    zg0, zg1, zg2 = coef_ref[7], coef_ref[8], coef_ref[9]
    s = s_ref[...]
    deg = deg_ref[...]
    mean = s / jnp.maximum(deg, 1.0)
    mx = jnp.where(deg > 0, mx_ref[...], 0.0)
    x_n = za0 * s + za1 * mean + za2 * mx
    xs = xself_ref[...]
    cc = (jnp.dot(xs, wc1_ref[...], preferred_element_type=jnp.float32)
          + jnp.dot(x_n, wc2_ref[...], preferred_element_type=jnp.float32) + bc_ref[...])
    h = zc0 * (xs + x_n) + zc1 * cc
    h = jnp.where(h >= 0, t_pos * h, t_neg * h)
    e3 = _ln(h, g_ref[...], bln_ref[...])
    e0, e1, e2 = e0_ref[...], e1_ref[...], e2_ref[...]
    t = (jnp.dot(e0, wa0_ref[...], preferred_element_type=jnp.float32)
         + jnp.dot(e1, wa1_ref[...], preferred_element_type=jnp.float32)
         + jnp.dot(e2, wa2_ref[...], preferred_element_type=jnp.float32)
         + jnp.dot(e3, wa3_ref[...], preferred_element_type=jnp.float32) + ba_ref[...])
    mmax = jnp.maximum(jnp.maximum(e0, e1), jnp.maximum(e2, e3))
    hagg = zg0 * e3 + zg1 * t + zg2 * mmax
    f1 = jnp.maximum(
        jnp.dot(hagg, w1_ref[...], preferred_element_type=jnp.float32) + b1_ref[...], 0.0)
    out_ref[...] = jnp.dot(f1, w2_ref[...], preferred_element_type=jnp.float32) + b2_ref[...]


def _row_out(n=1):
    sh = jax.ShapeDtypeStruct((N, D), jnp.float32)
    return [sh] * n


_W = lambda: _full_spec((D, D))
_B = lambda: _full_spec((1, D))
_C = lambda: pl.BlockSpec(memory_space=pltpu.SMEM)


def _tc_pre(x, wp, bp, ws, bs, wn, bn):
    return pl.pallas_call(
        _pre_body,
        grid=(N // BLK,),
        in_specs=[_row_spec(), _W(), _B(), _W(), _B(), _W(), _B()],
        out_specs=[_row_spec()] * 3,
        out_shape=_row_out(3),
    )(x, wp, bp, ws, bs, wn, bn)


def _tc_mid(coef, emb, xself, s, mx, deg, wc1, wc2, bc, g, bln,
            wl1, wl2, bl, ws, bs, wn, bn):
    return pl.pallas_call(
        _mid_body,
        grid=(N // BLK,),
        in_specs=[_C(), _row_spec(), _row_spec(), _row_spec(), _row_spec(),
                  pl.BlockSpec((BLK, 1), lambda i: (i, 0)),
                  _W(), _W(), _B(), _B(), _B(),
                  _W(), _W(), _B(),
                  _W(), _B(), _W(), _B()],
        out_specs=[_row_spec()] * 3,
        out_shape=_row_out(3),
    )(coef, emb, xself, s, mx, deg, wc1, wc2, bc, g, bln,
      wl1, wl2, bl, ws, bs, wn, bn)


def _tc_fin(coef, e0, e1, e2, xself, s, mx, deg, wc1, wc2, bc, g, bln,
            wa, ba, w1, b1, w2, b2):
    return pl.pallas_call(
        _fin_body,
        grid=(N // BLK,),
        in_specs=[_C(), _row_spec(), _row_spec(), _row_spec(), _row_spec(),
                  _row_spec(), _row_spec(),
                  pl.BlockSpec((BLK, 1), lambda i: (i, 0)),
                  _W(), _W(), _B(), _B(), _B(),
                  _W(), _W(), _W(), _W(), _B(),
                  _W(), _B(), _W(), _B()],
        out_specs=[_row_spec()],
        out_shape=_row_out(1),
    )(coef, e0, e1, e2, xself, s, mx, deg, wc1, wc2, bc, g, bln,
      wa[0], wa[1], wa[2], wa[3], ba, w1, b1, w2, b2)


from jax import lax
from jax.experimental.pallas import tpu_sc as plsc

NW = 32
NPT = 320
NROWS = 328
NPAD = NW * NPT
SCH = 2048
NFULL = E // SCH
TAIL = E - NFULL * SCH
FCH = 2048
SELCAP = FCH + SCH + 16
CAPW = E + FCH
CH = 128
TRASHW = NPT << 14

_SC_PARAMS = pltpu.CompilerParams(needs_layout_passes=False)


@functools.lru_cache(maxsize=None)
def _index_kernel():
    mesh = plsc.VectorSubcoreMesh(core_axis_name="c", subcore_axis_name="s")

    @functools.partial(
        pl.kernel, mesh=mesh,
        out_type=[jax.ShapeDtypeStruct((NW, CAPW), jnp.int32),
                  jax.ShapeDtypeStruct((NW, 16), jnp.int32)],
        compiler_params=_SC_PARAMS,
        scratch_types=[pltpu.VMEM((SELCAP,), jnp.int32),
                       pltpu.VMEM((SCH,), jnp.int32),
                       pltpu.VMEM((SCH,), jnp.int32),
                       pltpu.VMEM((16,), jnp.int32)])
    def idx_kernel(ei, lists, counts, sel, dbuf, sbuf, obuf):
        wid = lax.axis_index("s") * 2 + lax.axis_index("c")
        lo = wid * NPT
        trash = jnp.full((16,), TRASHW, jnp.int32)

        def fill_trash(g, _):
            sel[pl.ds(g * 16, 16)] = trash
            return 0

        lax.fori_loop(0, SELCAP // 16, fill_trash, 0)

        def scan_chunk(carry, off, ngroups):
            cnt, flushed = carry
            ne = ngroups * 16
            pltpu.sync_copy(ei.at[1, pl.ds(off, ne)], dbuf.at[pl.ds(0, ne)])
            pltpu.sync_copy(ei.at[0, pl.ds(off, ne)], sbuf.at[pl.ds(0, ne)])

            def group(g, cnt):
                d16 = dbuf[pl.ds(g * 16, 16)]
                s16 = sbuf[pl.ds(g * 16, 16)]
                m = (d16 >= lo) & (d16 < lo + NPT)
                word = ((d16 - lo) << 14) | s16
                plsc.store_compressed(sel.at[pl.ds(cnt, 16)], word, mask=m)
                pc = plsc.all_reduce_population_count(m)
                return cnt + pc[0]

            cnt = lax.fori_loop(0, ngroups, group, cnt, unroll=4)

            def do_flush(args):
                cnt, flushed = args
                fo = pl.multiple_of(flushed, FCH)
                pltpu.sync_copy(sel.at[pl.ds(0, FCH)],
                                lists.at[wid, pl.ds(fo, FCH)])

                def shift(g, _):
                    sel[pl.ds(g * 16, 16)] = sel[pl.ds(FCH + g * 16, 16)]
                    return 0

                lax.fori_loop(0, (SELCAP - FCH) // 16, shift, 0)

                def refill(g, _):
                    sel[pl.ds(g * 16, 16)] = trash
                    return 0

                lax.fori_loop((SELCAP - FCH) // 16, SELCAP // 16, refill, 0)
                return (cnt - FCH, flushed + FCH)

            return lax.cond(cnt >= FCH, do_flush, lambda a: a, (cnt, flushed))

        carry = (jnp.int32(0), jnp.int32(0))

        def chunk_body(c, carry):
            return scan_chunk(carry, pl.multiple_of(c * SCH, SCH), SCH // 16)

        carry = lax.fori_loop(0, NFULL, chunk_body, carry)
        carry = scan_chunk(carry, NFULL * SCH, TAIL // 16)
        cnt, flushed = carry
        fo = pl.multiple_of(flushed, FCH)
        pltpu.sync_copy(sel.at[pl.ds(0, FCH)], lists.at[wid, pl.ds(fo, FCH)])
        obuf[...] = jnp.full((16,), 1, jnp.int32) * (flushed + cnt)
        pltpu.sync_copy(obuf, counts.at[wid])

    return idx_kernel


NFC = D // 16


@functools.lru_cache(maxsize=None)
def _agg_kernel(with_deg):
    mesh = plsc.VectorSubcoreMesh(core_axis_name="c", subcore_axis_name="s")
    outs = [jax.ShapeDtypeStruct((NPAD * D,), jnp.float32),
            jax.ShapeDtypeStruct((NPAD * D,), jnp.float32)]
    if with_deg:
        outs.append(jax.ShapeDtypeStruct((NPAD,), jnp.float32))
    scr = [pltpu.VMEM((NROWS * D,), jnp.float32),
           pltpu.VMEM((NROWS * D,), jnp.float32),
           pltpu.VMEM((336,), jnp.float32),
           pltpu.VMEM((336,), jnp.int32),
           pltpu.VMEM((CH,), jnp.int32), pltpu.VMEM((CH,), jnp.int32),
           pltpu.VMEM((CH,), jnp.int32), pltpu.VMEM((CH,), jnp.int32),
           pltpu.VMEM((CH,), jnp.int32), pltpu.VMEM((CH,), jnp.int32),
           pltpu.VMEM((CH, D), jnp.float32), pltpu.VMEM((CH, D), jnp.float32),
           pltpu.VMEM((16,), jnp.int32),
           pltpu.SemaphoreType.DMA, pltpu.SemaphoreType.DMA]

    @functools.partial(pl.kernel, mesh=mesh, out_type=outs,
                       compiler_params=_SC_PARAMS, scratch_types=scr)
    def agg(xn, lists, counts, *refs):
        if with_deg:
            s_hbm, m_hbm, deg_hbm = refs[:3]
            rest = refs[3:]
        else:
            s_hbm, m_hbm = refs[:2]
            deg_hbm = None
            rest = refs[2:]
        (acc_s, acc_m, degacc, tmpi, wbuf0, wbuf1, sbuf0, sbuf1, kbuf0,
         kbuf1, rows0, rows1, cbuf, sem0, sem1) = rest
        bufs = ((wbuf0, sbuf0, kbuf0, rows0, sem0),
                (wbuf1, sbuf1, kbuf1, rows1, sem1))
        wid = lax.axis_index("s") * 2 + lax.axis_index("c")
        zeros = jnp.zeros((16,), jnp.float32)
        neg = jnp.full((16,), -3.0e38, jnp.float32)
        pltpu.sync_copy(counts.at[wid], cbuf)
        cnt = cbuf[...][0]
        nch = (cnt + CH - 1) // CH

        def init(i, _):
            b = i * D
            for f in range(NFC):
                acc_s[pl.ds(b + f * 16, 16)] = zeros
                acc_m[pl.ds(b + f * 16, 16)] = neg
            return 0

        lax.fori_loop(0, NROWS, init, 0)
        if with_deg:
            def initd(i, _):
                degacc[pl.ds(i * 16, 16)] = zeros
                return 0
            lax.fori_loop(0, 336 // 16, initd, 0)

        def load_chunk(c, b):
            wbuf, sbuf, kbuf, rows, sem = bufs[b]
            off = pl.multiple_of(c * CH, CH)
            pltpu.sync_copy(lists.at[wid, pl.ds(off, CH)], wbuf)

            def unpack(g, _):
                w16 = wbuf[pl.ds(g * 16, 16)]
                sbuf[pl.ds(g * 16, 16)] = w16 & 16383
                kbuf[pl.ds(g * 16, 16)] = lax.shift_right_logical(w16, 14)
                return 0

            lax.fori_loop(0, CH // 16, unpack, 0, unroll=2)
            pltpu.async_copy(xn.at[sbuf], rows, sem)

        def accum(b):
            _, sbuf, kbuf, rows, sem = bufs[b]
            pltpu.make_async_copy(xn.at[sbuf], rows, sem).wait()

            ones = jnp.full((16,), 1.0, jnp.float32)

            def group(g, _):
                dvec = kbuf[pl.ds(g * 16, 16)]
                base = g * 16
                if with_deg:
                    plsc.addupdate_scatter(degacc, [dvec], ones)
                for lane in range(16):
                    dl = dvec[lane]
                    db = dl * D
                    for f in range(NFC):
                        fs = pl.ds(db + f * 16, 16)
                        r = rows[base + lane, pl.ds(f * 16, 16)]
                        acc_s[fs] = acc_s[fs] + r
                        acc_m[fs] = jnp.maximum(acc_m[fs], r)
                return 0

            lax.fori_loop(0, CH // 16, group, 0)

        pl.when(nch > 0)(lambda: load_chunk(0, 0))

        def pair(k2, _):
            c0 = k2 * 2
            pl.when(c0 + 1 < nch)(lambda: load_chunk(c0 + 1, 1))
            accum(0)
            pl.when(c0 + 2 < nch)(lambda: load_chunk(c0 + 2, 0))
            pl.when(c0 + 1 < nch)(lambda: accum(1))
            return 0

        lax.fori_loop(0, (nch + 1) // 2, pair, 0)
        ro = pl.multiple_of(wid * (NPT * D), NPT * D)
        pltpu.sync_copy(acc_s.at[pl.ds(0, NPT * D)], s_hbm.at[pl.ds(ro, NPT * D)])
        pltpu.sync_copy(acc_m.at[pl.ds(0, NPT * D)], m_hbm.at[pl.ds(ro, NPT * D)])
        if with_deg:
            rd = pl.multiple_of(wid * NPT, NPT)
            pltpu.sync_copy(degacc.at[pl.ds(0, NPT)],
                            deg_hbm.at[pl.ds(rd, NPT)])

    return agg


def kernel(x, edge_index, batch, params):
    src, dst = edge_index[0], edge_index[1]
    Z = params["Z"]
    P = params

    def b2(p):
        return p["b"].reshape(1, D)

    h0, xself, xn = _tc_pre(
        x, P["preprocess"]["W"], b2(P["preprocess"]),
        P["linears_self"][0]["W"], b2(P["linears_self"][0]),
        P["linears"][0]["W"], b2(P["linears"][0]))

    lists, counts = _index_kernel()(edge_index)
    deg = None
    emb = [h0]
    for i in range(LAYERS):
        if i == 0:
            s, mx, degp = _agg_kernel(True)(xn, lists, counts)
            deg = degp[:N].reshape(N, 1)
        else:
            s, mx = _agg_kernel(False)(xn, lists, counts)
        s = s.reshape(NPAD, D)[:N]
        mx = mx.reshape(NPAD, D)[:N]
        za = Z["agg"][i]
        zc = Z["combine"][i]
        zt = Z["act"][i]
        a = P["prelu"][i]
        wc = P["combine_merger"][i]["W"]
        wc1, wc2 = wc[:D], wc[D:]
        ln = P["layer_norms"][i]
        if i < LAYERS - 1:
            zl = Z["layer_connect"][i]
            coef = jnp.stack([za[0], za[1], za[2], zc[0], zc[1],
                              zt[0] + zt[1], zt[1] * a, zl[0], zl[1], zl[2]])
            wl = P["layer_connect_merger"][i]["W"]
            e, xself, xn = _tc_mid(
                coef, emb[i], xself, s, mx, deg,
                wc1, wc2, b2(P["combine_merger"][i]),
                ln["g"].reshape(1, D), ln["b"].reshape(1, D),
                wl[:D], wl[D:], b2(P["layer_connect_merger"][i]),
                P["linears_self"][i + 1]["W"], b2(P["linears_self"][i + 1]),
                P["linears"][i + 1]["W"], b2(P["linears"][i + 1]))
            emb.append(e)
        else:
            zg = Z["layer_agg"][0]
            coef = jnp.stack([za[0], za[1], za[2], zc[0], zc[1],
                              zt[0] + zt[1], zt[1] * a, zg[0], zg[1], zg[2]])
            wa = P["layer_agg_merger"]["W"]
            was = [wa[k * D:(k + 1) * D] for k in range(LAYERS + 1)]
            (out,) = _tc_fin(
                coef, emb[0], emb[1], emb[2], xself, s, mx, deg,
                wc1, wc2, b2(P["combine_merger"][i]),
                ln["g"].reshape(1, D), ln["b"].reshape(1, D),
                was, b2(P["layer_agg_merger"]),
                P["ffn1"]["W"], b2(P["ffn1"]),
                P["ffn2"]["W"], b2(P["ffn2"]))
    return out
